# baseline (speedup 1.0000x reference)
import jax
import jax.numpy as jnp
from jax import lax
from jax.experimental import pallas as pl
from jax.experimental.pallas import tpu as pltpu
from jax.experimental.pallas import tpu_sc as plsc

_NC, _NS = 2, 16
_NW = _NC * _NS
_L = 16


def _wid():
    return lax.axis_index("s") * _NC + lax.axis_index("c")


def _requad_body(emb_t, emb_tail, tbl_q, in_v, out_a, out_b,
                 is0, is1, os0, os1):
    wid = _wid()
    rows16 = lax.iota(jnp.int32, _L)
    iss = (is0, is1)
    oss = (os0, os1)
    outs = (out_a, out_b)

    def fire_in(sb, s):
        pltpu.async_copy(emb_t.at[:, pl.ds(sb * 512, 512)],
                         in_v.at[s, :, pl.ds(0, 512)], iss[s])

    def wait_in(s):
        pltpu.make_async_copy(emb_t.at[:, pl.ds(0, 512)],
                              in_v.at[s, :, pl.ds(0, 512)], iss[s]).wait()

    l_div4 = rows16 >> 2
    l_mod4_32 = (rows16 & 3) * 32

    def transpose(s):
        @plsc.parallel_loop(0, 32, 1, unroll=2)
        def _(t):
            row = l_div4 + 4 * t
            for f in range(32):
                vals = in_v[s, f, pl.ds(16 * t, _L)]
                plsc.store_scatter(outs[s], [row, l_mod4_32 + f], vals)

    def fire_out(sb, s):
        pltpu.async_copy(outs[s], tbl_q.at[pl.ds(sb * 128, 128)], oss[s])

    def wait_out(s):
        pltpu.make_async_copy(outs[s], tbl_q.at[pl.ds(0, 128)],
                              oss[s]).wait()

    def blk(k):
        return k * _NW + wid

    fire_in(blk(0), 0)
    fire_in(blk(1), 1)

    def pair(p, carry):
        k0 = 2 * p
        wait_in(0)

        @pl.when(p > 0)
        def _():
            wait_out(0)

        transpose(0)
        fire_out(blk(k0), 0)

        @pl.when(k0 + 2 < 61)
        def _():
            fire_in(blk(k0 + 2), 0)

        wait_in(1)

        @pl.when(p > 0)
        def _():
            wait_out(1)

        transpose(1)
        fire_out(blk(k0 + 1), 1)

        @pl.when(k0 + 3 < 61)
        def _():
            fire_in(blk(k0 + 3), 1)

        return carry

    lax.fori_loop(0, 30, pair, 0)
    wait_in(0)
    wait_out(0)
    transpose(0)
    fire_out(blk(60), 0)
    wait_out(0)
    wait_out(1)

    @pl.when(wid == 0)
    def _():
        fire_in(1952, 0)
        wait_in(0)
        transpose(0)
        fire_out(1952, 0)
        wait_out(0)

    @pl.when(wid == 1)
    def _():
        pltpu.async_copy(emb_tail, tbl_q.at[pl.ds(7812 * 32, 16)], os0).wait()


def _gather_body(tbl_q, idx_t, out_t, idx_v, q_v, r_v, g_v, o_v,
                 isem, gs0, gs1, os0, os1):
    wid = _wid()
    rows16 = lax.iota(jnp.int32, _L)
    gss = (gs0, gs1)
    oss = (os0, os1)

    def prep(j, s):
        @plsc.parallel_loop(0, 8, 1, unroll=2)
        def _(b):
            v = idx_v[j, pl.ds(16 * b, _L)]
            q_v[s, pl.ds(16 * b, _L)] = v >> 2
            r_v[s, pl.ds(16 * b, _L)] = (v & 3) * 32

    def fire(s):
        pltpu.async_copy(tbl_q.at[q_v.at[s]],
                         g_v.at[s, :, pl.ds(0, 128)], gss[s])

    def drain(s):
        pltpu.make_async_copy(tbl_q.at[pl.ds(0, 128)],
                              g_v.at[s, :, pl.ds(0, 128)], gss[s]).wait()

    def extract(s, t):
        bases = [r_v[s, pl.ds(16 * lb, _L)] for lb in range(8)]

        @plsc.parallel_loop(0, 32, 1, unroll=2)
        def _(f):
            for lb in range(8):
                rr = rows16 + 16 * lb
                vals = plsc.load_gather(g_v.at[s], [rr, bases[lb] + f])
                o_v[t, f, pl.ds(16 * lb, _L)] = vals

    def wb(j, c, t):
        pltpu.async_copy(o_v.at[t], out_t.at[j, :, pl.ds(c * 128, 128)],
                         oss[t])

    def wb_wait(t):
        pltpu.make_async_copy(o_v.at[t], out_t.at[0, :, pl.ds(0, 128)],
                              oss[t]).wait()

    def do_cblock(cc, carry):
        c = cc * _NW + wid
        pltpu.async_copy(idx_t.at[:, pl.ds(c * 128, 128)], idx_v, isem)
        pltpu.make_async_copy(idx_t.at[:, pl.ds(0, 128)], idx_v, isem).wait()
        prep(0, 0)
        fire(0)

        def pair(p, carry2):
            j0 = 2 * p
            prep(j0 + 1, 1)
            fire(1)
            drain(0)

            @pl.when((cc > 0) | (p > 0))
            def _():
                wb_wait(0)

            extract(0, 0)
            wb(j0, c, 0)

            @pl.when(p < 24)
            def _():
                prep(j0 + 2, 0)
                fire(0)

            drain(1)

            @pl.when((cc > 0) | (p > 0))
            def _():
                wb_wait(1)

            extract(1, 1)
            wb(j0 + 1, c, 1)
            return carry2

        lax.fori_loop(0, 25, pair, 0)
        return carry

    lax.fori_loop(0, 16384 // 128 // _NW, do_cblock, 0)
    wb_wait(0)
    wb_wait(1)


def kernel(inputs, embedding):
    if inputs.shape[-1] == 1:
        inputs = jnp.squeeze(inputs, axis=-1)
    n, m = inputs.shape
    dim = embedding.shape[1]

    emb_t = jnp.transpose(embedding)
    idx_t = jnp.transpose(inputs).astype(jnp.int32)
    n_tail = embedding.shape[0] % 512
    emb_tail = jnp.reshape(
        lax.slice(embedding, (embedding.shape[0] - n_tail, 0),
                  (embedding.shape[0], dim)), (n_tail * dim // 128, 128))
    mesh = plsc.VectorSubcoreMesh(core_axis_name="c", subcore_axis_name="s")
    params = pltpu.CompilerParams(use_tc_tiling_on_sc=True,
                                  needs_layout_passes=False)

    use_requad_kernel = False
    if use_requad_kernel:
        tbl_q = pl.kernel(
            _requad_body,
            out_type=jax.ShapeDtypeStruct((250000, 128), jnp.float32),
            mesh=mesh,
            compiler_params=params,
            scratch_types=[
                pltpu.VMEM((2, dim, 513), jnp.float32),
                pltpu.VMEM((128, 128), jnp.float32),
                pltpu.VMEM((128, 128), jnp.float32),
            ] + [pltpu.SemaphoreType.DMA] * 4,
        )(emb_t, emb_tail)
    else:
        tbl_q = jnp.reshape(embedding, (250000, 128))

    out_t = pl.kernel(
        _gather_body,
        out_type=jax.ShapeDtypeStruct((m, dim, n), jnp.float32),
        mesh=mesh,
        compiler_params=params,
        scratch_types=[
            pltpu.VMEM((m, 128), jnp.int32),
            pltpu.VMEM((2, 128), jnp.int32),
            pltpu.VMEM((2, 128), jnp.int32),
            pltpu.VMEM((2, 128, 129), jnp.float32),
            pltpu.VMEM((2, dim, 128), jnp.float32),
        ] + [pltpu.SemaphoreType.DMA] * 5,
    )(tbl_q, idx_t)

    return jnp.transpose(out_t, (2, 0, 1))


# speedup vs baseline: 1.1565x; 1.1565x over previous
import jax
import jax.numpy as jnp
from jax import lax
from jax.experimental import pallas as pl
from jax.experimental.pallas import tpu as pltpu
from jax.experimental.pallas import tpu_sc as plsc

_NC, _NS = 2, 16
_NW = _NC * _NS
_L = 16


def _wid():
    return lax.axis_index("s") * _NC + lax.axis_index("c")


def _requad_body(emb_t, emb_tail, tbl_q, in_v, out_v, is0, is1, os0, os1):
    wid = _wid()
    rows16 = lax.iota(jnp.int32, _L)
    iss = (is0, is1)
    oss = (os0, os1)

    def fire_in(c, s):
        pltpu.async_copy(emb_t.at[:, pl.ds(c * 128, 128)], in_v.at[s], iss[s])

    def wait_in(s):
        pltpu.make_async_copy(emb_t.at[:, pl.ds(0, 128)], in_v.at[s],
                              iss[s]).wait()

    def transpose(s):
        @plsc.parallel_loop(0, 32, 1, unroll=4)
        def _(a):
            for b in range(4):
                col = jnp.broadcast_to(4 * a + b, (_L,)).astype(jnp.int32)
                for h in range(2):
                    vals = plsc.load_gather(in_v.at[s],
                                            [rows16 + 16 * h, col])
                    out_v[s, a, pl.ds(32 * b + 16 * h, _L)] = vals

    def fire_out(c, s):
        pltpu.async_copy(out_v.at[s], tbl_q.at[pl.ds(c * 32, 32)], oss[s])

    def wait_out(s):
        pltpu.make_async_copy(out_v.at[s], tbl_q.at[pl.ds(0, 32)],
                              oss[s]).wait()

    def blk(k):
        return k * _NW + wid

    fire_in(blk(0), 0)

    def pair(p, carry):
        fire_in(blk(2 * p + 1), 1)
        wait_in(0)

        @pl.when(p > 0)
        def _():
            wait_out(0)

        transpose(0)
        fire_out(blk(2 * p), 0)

        @pl.when(p < 121)
        def _():
            fire_in(blk(2 * p + 2), 0)

        wait_in(1)

        @pl.when(p > 0)
        def _():
            wait_out(1)

        transpose(1)
        fire_out(blk(2 * p + 1), 1)
        return carry

    lax.fori_loop(0, 122, pair, 0)
    wait_out(0)
    wait_out(1)

    @pl.when(wid < 4)
    def _():
        fire_in(7808 + wid, 0)
        wait_in(0)
        transpose(0)
        fire_out(7808 + wid, 0)
        wait_out(0)

    @pl.when(wid == 4)
    def _():
        pltpu.async_copy(emb_tail, tbl_q.at[pl.ds(7812 * 32, 16)], os0).wait()


def _gather_body(tbl_q, idx_t, out_t, idx_v, q_v, r_v, g_v, o_v,
                 isem, gs0, gs1, os0, os1):
    wid = _wid()
    rows16 = lax.iota(jnp.int32, _L)
    gss = (gs0, gs1)
    oss = (os0, os1)

    def prep(j, s):
        @plsc.parallel_loop(0, 8, 1, unroll=2)
        def _(b):
            v = idx_v[j, pl.ds(16 * b, _L)]
            q_v[s, pl.ds(16 * b, _L)] = v >> 2
            r_v[s, pl.ds(16 * b, _L)] = (v & 3) * 32

    def fire(s):
        pltpu.async_copy(tbl_q.at[q_v.at[s]], g_v.at[s], gss[s])

    def drain(s):
        pltpu.make_async_copy(tbl_q.at[pl.ds(0, 128)], g_v.at[s],
                              gss[s]).wait()

    def extract(s, t):
        bases = [r_v[s, pl.ds(16 * lb, _L)] for lb in range(8)]

        @plsc.parallel_loop(0, 32, 1, unroll=2)
        def _(f):
            for lb in range(8):
                rr = rows16 + 16 * lb
                vals = plsc.load_gather(g_v.at[s], [rr, bases[lb] + f])
                o_v[t, f, pl.ds(16 * lb, _L)] = vals

    def wb(j, c, t):
        pltpu.async_copy(o_v.at[t], out_t.at[j, :, pl.ds(c * 128, 128)],
                         oss[t])

    def wb_wait(t):
        pltpu.make_async_copy(o_v.at[t], out_t.at[0, :, pl.ds(0, 128)],
                              oss[t]).wait()

    def do_cblock(cc, carry):
        c = cc * _NW + wid
        pltpu.async_copy(idx_t.at[:, pl.ds(c * 128, 128)], idx_v, isem)
        pltpu.make_async_copy(idx_t.at[:, pl.ds(0, 128)], idx_v, isem).wait()
        prep(0, 0)
        fire(0)

        def pair(p, carry2):
            j0 = 2 * p
            prep(j0 + 1, 1)
            fire(1)
            drain(0)

            @pl.when((cc > 0) | (p > 0))
            def _():
                wb_wait(0)

            extract(0, 0)
            wb(j0, c, 0)

            @pl.when(p < 24)
            def _():
                prep(j0 + 2, 0)
                fire(0)

            drain(1)

            @pl.when((cc > 0) | (p > 0))
            def _():
                wb_wait(1)

            extract(1, 1)
            wb(j0 + 1, c, 1)
            return carry2

        lax.fori_loop(0, 25, pair, 0)
        return carry

    lax.fori_loop(0, 16384 // 128 // _NW, do_cblock, 0)
    wb_wait(0)
    wb_wait(1)


def kernel(inputs, embedding):
    if inputs.shape[-1] == 1:
        inputs = jnp.squeeze(inputs, axis=-1)
    n, m = inputs.shape
    dim = embedding.shape[1]

    emb_t = jnp.transpose(embedding)
    idx_t = jnp.transpose(inputs).astype(jnp.int32)
    n_tail = embedding.shape[0] % 512
    emb_tail = jnp.reshape(
        lax.slice(embedding, (embedding.shape[0] - n_tail, 0),
                  (embedding.shape[0], dim)), (n_tail * dim // 128, 128))
    mesh = plsc.VectorSubcoreMesh(core_axis_name="c", subcore_axis_name="s")
    params = pltpu.CompilerParams(use_tc_tiling_on_sc=True,
                                  needs_layout_passes=False)

    tbl_q = pl.kernel(
        _requad_body,
        out_type=jax.ShapeDtypeStruct((250000, 128), jnp.float32),
        mesh=mesh,
        compiler_params=params,
        scratch_types=[
            pltpu.VMEM((2, dim, 128), jnp.float32),
            pltpu.VMEM((2, 32, 128), jnp.float32),
        ] + [pltpu.SemaphoreType.DMA] * 4,
    )(emb_t, emb_tail)

    out_t = pl.kernel(
        _gather_body,
        out_type=jax.ShapeDtypeStruct((m, dim, n), jnp.float32),
        mesh=mesh,
        compiler_params=params,
        scratch_types=[
            pltpu.VMEM((m, 128), jnp.int32),
            pltpu.VMEM((2, 128), jnp.int32),
            pltpu.VMEM((2, 128), jnp.int32),
            pltpu.VMEM((2, 128, 128), jnp.float32),
            pltpu.VMEM((2, dim, 128), jnp.float32),
        ] + [pltpu.SemaphoreType.DMA] * 5,
    )(tbl_q, idx_t)

    return jnp.transpose(out_t, (2, 0, 1))
